# baseline (speedup 1.0000x reference)
import functools

import jax
import jax.numpy as jnp
from jax import lax
from jax.experimental import pallas as pl
from jax.experimental.pallas import tpu as pltpu
from jax.experimental.pallas import tpu_sc as plsc

N = 16384
H = N // 2
DIN = 768
HID = 128
F = 256
K = 1024
TILE = 512
GRID = H // TILE


def _row_sq_norm(z):
    zz = z * z
    acc = zz[:, 0:8]
    for k in range(1, 32):
        acc = acc + zz[:, 8 * k:8 * k + 8]
    a = [acc[:, s:s + 1] for s in range(8)]
    return ((a[0] + a[4]) + (a[2] + a[6])) + ((a[1] + a[5]) + (a[3] + a[7]))


def _encode_common(x_ref, g_ref, w1_ref, b1_ref, w2_ref, b2_ref, cbt_ref,
                   csq_ref, nd_ref, enc_ref, z_ref, idx_ref, cnt_ref):
    i = pl.program_id(0)

    @pl.when(i == 0)
    def _init():
        cnt_ref[...] = jnp.zeros_like(cnt_ref)

    x = x_ref[...]
    h = jnp.maximum(jnp.dot(x, w1_ref[...]) + b1_ref[...], 0.0)
    z = jnp.dot(h, w2_ref[...]) + b2_ref[...]
    z_ref[...] = z

    zsq = _row_sq_norm(z)
    csq = csq_ref[...]
    zc = jnp.dot(z, cbt_ref[...])
    dist = (zsq + csq) - 2.0 * zc
    nd = -dist
    nd_ref[...] = nd
    logits = nd / 0.1
    score = g_ref[...] + logits
    idx = jnp.argmax(score, axis=1)
    idx_ref[...] = idx.reshape(TILE, 1)

    iota = jax.lax.broadcasted_iota(jnp.int32, (TILE, K), 1)
    enc = (idx[:, None] == iota).astype(jnp.float32)
    enc_ref[...] = enc
    cnt_ref[...] += jnp.sum(enc, axis=0, keepdims=True)


def _encode0_body(*refs):
    _encode_common(*refs)


def _encode1_body(x_ref, g_ref, w1_ref, b1_ref, w2_ref, b2_ref, cbt_ref,
                  csq_ref, nd_buf, enc_buf, nd_ref, enc_ref, z_ref, idx_ref,
                  cnt_ref):
    del nd_buf, enc_buf
    _encode_common(x_ref, g_ref, w1_ref, b1_ref, w2_ref, b2_ref, cbt_ref,
                   csq_ref, nd_ref, enc_ref, z_ref, idx_ref, cnt_ref)


def _fin0_body(z_ref, q_ref, qst_ref, sse_ref):
    i = pl.program_id(0)

    @pl.when(i == 0)
    def _init():
        sse_ref[...] = jnp.zeros_like(sse_ref)

    z = z_ref[...]
    d = q_ref[...] - z
    qst_ref[...] = z + d
    sse_ref[...] += jnp.sum(d * d).reshape(1, 1)


def _fin1_body(z_ref, q_ref, cnt0_ref, cnt1_ref, sse0_ref, qst_buf,
               qst_ref, loss_ref, ppl_ref, sse_ref):
    del qst_buf
    i = pl.program_id(0)

    @pl.when(i == 0)
    def _init():
        sse_ref[...] = sse0_ref[...]

    z = z_ref[...]
    d = q_ref[...] - z
    qst_ref[...] = z + d
    sse_ref[...] += jnp.sum(d * d).reshape(1, 1)

    @pl.when(i == GRID - 1)
    def _fin():
        loss_ref[...] = 2.0 * (sse_ref[...] / float(N * F))
        avg = (cnt0_ref[...] + cnt1_ref[...]) / float(N)
        ppl_ref[...] = jnp.exp(-jnp.sum(avg * jnp.log(avg + 1e-10))).reshape(1, 1)


def _make_sc_gather():
    info = plsc.get_sparse_core_info()
    nw = info.num_cores * info.num_subcores
    b_per_w = H // nw
    ch = 128
    mesh = plsc.VectorSubcoreMesh(core_axis_name="c", subcore_axis_name="s")

    @functools.partial(
        pl.kernel, mesh=mesh,
        out_type=jax.ShapeDtypeStruct((H, F), jnp.float32),
        scratch_types=[
            pltpu.VMEM((ch,), jnp.int32),
            pltpu.VMEM((ch, F), jnp.float32),
            pltpu.SemaphoreType.DMA,
        ],
    )
    def gather(cb_hbm, idx_hbm, q_hbm, idx_v, rows_v, sem):
        wid = lax.axis_index("s") * info.num_cores + lax.axis_index("c")
        base = wid * b_per_w
        for c in range(b_per_w // ch):
            off = base + c * ch
            pltpu.sync_copy(idx_hbm.at[pl.ds(off, ch)], idx_v)
            pltpu.async_copy(cb_hbm.at[idx_v], rows_v, sem).wait()
            pltpu.sync_copy(rows_v, q_hbm.at[pl.ds(off, ch)])

    return gather


_SC_GATHER = _make_sc_gather()

_GUMBEL = jax.random.gumbel(jax.random.key(42), (N, K), jnp.float32)

_lo_spec = lambda w: pl.BlockSpec((TILE, w), lambda i: (i, 0))
_hi_spec = lambda w: pl.BlockSpec((TILE, w), lambda i: (i + GRID, 0))
_full_spec = lambda a, b: pl.BlockSpec((a, b), lambda i: (0, 0))
_any_spec = pl.BlockSpec(memory_space=pl.ANY)

_ENC_IN_SPECS = [
    _lo_spec(DIN),
    _lo_spec(K),
    _full_spec(DIN, HID),
    _full_spec(1, HID),
    _full_spec(HID, F),
    _full_spec(1, F),
    _full_spec(F, K),
    _full_spec(1, K),
]


def _enc_out(half_spec):
    return (
        (half_spec(K), half_spec(K), _lo_spec(F),
         pl.BlockSpec((TILE, 1), lambda i: (i, 0)), _full_spec(1, K)),
        (
            jax.ShapeDtypeStruct((N, K), jnp.float32),
            jax.ShapeDtypeStruct((N, K), jnp.float32),
            jax.ShapeDtypeStruct((H, F), jnp.float32),
            jax.ShapeDtypeStruct((H, 1), jnp.int32),
            jax.ShapeDtypeStruct((1, K), jnp.float32),
        ),
    )


_ENC0_OUT_SPECS, _ENC_OUT_SHAPES = _enc_out(_lo_spec)
_ENC1_OUT_SPECS, _ = _enc_out(_hi_spec)


def kernel(input_data, W1, b1, W2, b2, code_book):
    b1r = b1.reshape(1, HID)
    b2r = b2.reshape(1, F)
    cbt = code_book.T
    csq = jnp.sum(code_book**2, axis=1).reshape(1, K)

    nd, enc, z0, idx0, cnt0 = pl.pallas_call(
        _encode0_body,
        grid=(GRID,),
        in_specs=_ENC_IN_SPECS,
        out_specs=_ENC0_OUT_SPECS,
        out_shape=_ENC_OUT_SHAPES,
    )(input_data[:H], _GUMBEL[:H], W1, b1r, W2, b2r, cbt, csq)

    q0 = _SC_GATHER(code_book, idx0.reshape(H))

    nd, enc, z1, idx1, cnt1 = pl.pallas_call(
        _encode1_body,
        grid=(GRID,),
        in_specs=_ENC_IN_SPECS + [_any_spec, _any_spec],
        out_specs=_ENC1_OUT_SPECS,
        out_shape=_ENC_OUT_SHAPES,
        input_output_aliases={8: 0, 9: 1},
    )(input_data[H:], _GUMBEL[H:], W1, b1r, W2, b2r, cbt, csq, nd, enc)

    q1 = _SC_GATHER(code_book, idx1.reshape(H))

    qst, sse0 = pl.pallas_call(
        _fin0_body,
        grid=(GRID,),
        in_specs=[_lo_spec(F), _lo_spec(F)],
        out_specs=(_lo_spec(F), _full_spec(1, 1)),
        out_shape=(
            jax.ShapeDtypeStruct((N, F), jnp.float32),
            jax.ShapeDtypeStruct((1, 1), jnp.float32),
        ),
    )(z0, q0)

    qst, loss, ppl = pl.pallas_call(
        _fin1_body,
        grid=(GRID,),
        in_specs=[
            _lo_spec(F),
            _lo_spec(F),
            _full_spec(1, K),
            _full_spec(1, K),
            _full_spec(1, 1),
            _any_spec,
        ],
        out_specs=(_hi_spec(F), _full_spec(1, 1), _full_spec(1, 1)),
        out_shape=(
            jax.ShapeDtypeStruct((N, F), jnp.float32),
            jax.ShapeDtypeStruct((1, 1), jnp.float32),
            jax.ShapeDtypeStruct((1, 1), jnp.float32),
        ),
        scratch_shapes=[pltpu.VMEM((1, 1), jnp.float32)],
        input_output_aliases={5: 0},
    )(z1, q1, cnt0, cnt1, sse0, qst)

    return (loss.reshape(()), qst, ppl.reshape(()), enc, nd)


# speedup vs baseline: 1.1978x; 1.1978x over previous
import functools

import jax
import jax.numpy as jnp
from jax import lax
from jax.experimental import pallas as pl
from jax.experimental.pallas import tpu as pltpu
from jax.experimental.pallas import tpu_sc as plsc

N = 16384
DIN = 768
HID = 128
F = 256
K = 1024
TILE = 512
GRID = N // TILE
FTILE = 2048
FGRID = N // FTILE


def _row_sq_norm(z):
    zz = z * z
    acc = zz[:, 0:8]
    for k in range(1, 32):
        acc = acc + zz[:, 8 * k:8 * k + 8]
    a = [acc[:, s:s + 1] for s in range(8)]
    return ((a[0] + a[4]) + (a[2] + a[6])) + ((a[1] + a[5]) + (a[3] + a[7]))


def _encode_body(x_ref, g_ref, w1_ref, b1_ref, w2_ref, b2_ref, cbt_ref,
                 csq_ref, nd_ref, enc_ref, z_ref, idx_ref, cnt_ref):
    i = pl.program_id(0)

    @pl.when(i == 0)
    def _init():
        cnt_ref[...] = jnp.zeros_like(cnt_ref)

    x = x_ref[...]
    h = jnp.maximum(jnp.dot(x, w1_ref[...]) + b1_ref[...], 0.0)
    z = jnp.dot(h, w2_ref[...]) + b2_ref[...]
    z_ref[...] = z

    zsq = _row_sq_norm(z)
    csq = csq_ref[...]
    zc = jnp.dot(z, cbt_ref[...])
    dist = (zsq + csq) - 2.0 * zc
    nd = -dist
    nd_ref[...] = nd
    logits = nd / 0.1
    score = g_ref[...] + logits
    idx = jnp.argmax(score, axis=1)
    idx_ref[...] = idx.reshape(TILE, 1)

    iota = jax.lax.broadcasted_iota(jnp.int32, (TILE, K), 1)
    enc = (idx[:, None] == iota).astype(jnp.float32)
    enc_ref[...] = enc
    cnt_ref[...] += jnp.sum(enc, axis=0, keepdims=True)


def _finalize_body(z_ref, q_ref, cnt_ref, qst_ref, loss_ref, ppl_ref, sse_ref):
    i = pl.program_id(0)

    @pl.when(i == 0)
    def _init():
        sse_ref[...] = jnp.zeros_like(sse_ref)

    z = z_ref[...]
    d = q_ref[...] - z
    qst_ref[...] = z + d
    dd = d * d
    acc = sse_ref[...]
    for s in range(FTILE // TILE):
        acc = acc + jnp.sum(dd[s * TILE:(s + 1) * TILE, :]).reshape(1, 1)
    sse_ref[...] = acc

    @pl.when(i == FGRID - 1)
    def _fin():
        loss_ref[...] = 2.0 * (sse_ref[...] / float(N * F))
        avg = cnt_ref[...] / float(N)
        ppl_ref[...] = jnp.exp(-jnp.sum(avg * jnp.log(avg + 1e-10))).reshape(1, 1)


def _make_sc_gather():
    info = plsc.get_sparse_core_info()
    nw = info.num_cores * info.num_subcores
    b_per_w = N // nw
    ch = 128
    mesh = plsc.VectorSubcoreMesh(core_axis_name="c", subcore_axis_name="s")

    @functools.partial(
        pl.kernel, mesh=mesh,
        out_type=jax.ShapeDtypeStruct((N, F), jnp.float32),
        scratch_types=[
            pltpu.VMEM((ch,), jnp.int32),
            pltpu.VMEM((ch, F), jnp.float32),
            pltpu.SemaphoreType.DMA,
        ],
    )
    def gather(cb_hbm, idx_hbm, q_hbm, idx_v, rows_v, sem):
        wid = lax.axis_index("s") * info.num_cores + lax.axis_index("c")
        base = wid * b_per_w
        for c in range(b_per_w // ch):
            off = base + c * ch
            pltpu.sync_copy(idx_hbm.at[pl.ds(off, ch)], idx_v)
            pltpu.async_copy(cb_hbm.at[idx_v], rows_v, sem).wait()
            pltpu.sync_copy(rows_v, q_hbm.at[pl.ds(off, ch)])

    return gather


_SC_GATHER = _make_sc_gather()

_GUMBEL = jax.random.gumbel(jax.random.key(42), (N, K), jnp.float32)


def kernel(input_data, W1, b1, W2, b2, code_book):
    g = _GUMBEL
    b1r = b1.reshape(1, HID)
    b2r = b2.reshape(1, F)
    cbt = code_book.T
    csq = jnp.sum(code_book**2, axis=1).reshape(1, K)

    row_spec = lambda w: pl.BlockSpec((TILE, w), lambda i: (i, 0))
    fin_spec = lambda w: pl.BlockSpec((FTILE, w), lambda i: (i, 0))
    full_spec = lambda a, b: pl.BlockSpec((a, b), lambda i: (0, 0))

    nd, enc, z, idx, cnt = pl.pallas_call(
        _encode_body,
        grid=(GRID,),
        in_specs=[
            row_spec(DIN),
            row_spec(K),
            full_spec(DIN, HID),
            full_spec(1, HID),
            full_spec(HID, F),
            full_spec(1, F),
            full_spec(F, K),
            full_spec(1, K),
        ],
        out_specs=(
            row_spec(K),
            row_spec(K),
            row_spec(F),
            pl.BlockSpec((TILE, 1), lambda i: (i, 0)),
            full_spec(1, K),
        ),
        out_shape=(
            jax.ShapeDtypeStruct((N, K), jnp.float32),
            jax.ShapeDtypeStruct((N, K), jnp.float32),
            jax.ShapeDtypeStruct((N, F), jnp.float32),
            jax.ShapeDtypeStruct((N, 1), jnp.int32),
            jax.ShapeDtypeStruct((1, K), jnp.float32),
        ),
    )(input_data, g, W1, b1r, W2, b2r, cbt, csq)

    q = _SC_GATHER(code_book, idx.reshape(N))

    qst, loss, ppl = pl.pallas_call(
        _finalize_body,
        grid=(FGRID,),
        in_specs=[
            fin_spec(F),
            fin_spec(F),
            full_spec(1, K),
        ],
        out_specs=(
            fin_spec(F),
            full_spec(1, 1),
            full_spec(1, 1),
        ),
        out_shape=(
            jax.ShapeDtypeStruct((N, F), jnp.float32),
            jax.ShapeDtypeStruct((1, 1), jnp.float32),
            jax.ShapeDtypeStruct((1, 1), jnp.float32),
        ),
        scratch_shapes=[
            pltpu.VMEM((1, 1), jnp.float32),
        ],
    )(z, q, cnt)

    return (loss.reshape(()), qst, ppl.reshape(()), enc, nd)


# speedup vs baseline: 1.2053x; 1.0063x over previous
import functools

import jax
import jax.numpy as jnp
from jax import lax
from jax.experimental import pallas as pl
from jax.experimental.pallas import tpu as pltpu
from jax.experimental.pallas import tpu_sc as plsc

N = 16384
DIN = 768
HID = 128
F = 256
K = 1024
TILE = 512
GRID = N // TILE
FTILE = 2048
FGRID = N // FTILE


def _row_sq_norm(z):
    zz = z * z
    acc = zz[:, 0:8]
    for k in range(1, 32):
        acc = acc + zz[:, 8 * k:8 * k + 8]
    a = [acc[:, s:s + 1] for s in range(8)]
    return ((a[0] + a[4]) + (a[2] + a[6])) + ((a[1] + a[5]) + (a[3] + a[7]))


def _encode_body(x_ref, g_ref, w1_ref, b1_ref, w2_ref, b2_ref, cbt_ref,
                 csq_ref, nd_ref, enc_ref, z_ref, idx_ref, cnt_ref):
    i = pl.program_id(0)

    @pl.when(i == 0)
    def _init():
        cnt_ref[...] = jnp.zeros_like(cnt_ref)

    x = x_ref[...]
    h = jnp.maximum(jnp.dot(x, w1_ref[...]) + b1_ref[...], 0.0)
    z = jnp.dot(h, w2_ref[...]) + b2_ref[...]
    z_ref[...] = z

    zsq = _row_sq_norm(z)
    csq = csq_ref[...]
    zc = jnp.dot(z, cbt_ref[...])
    dist = (zsq + csq) - 2.0 * zc
    nd = -dist
    nd_ref[...] = nd
    logits = nd / 0.1
    score = g_ref[...] + logits
    idx = jnp.argmax(score, axis=1)
    idx_ref[...] = idx.reshape(TILE, 1)

    iota = jax.lax.broadcasted_iota(jnp.int32, (TILE, K), 1)
    enc = (idx[:, None] == iota).astype(jnp.float32)
    enc_ref[...] = enc
    cnt_ref[...] += jnp.sum(enc, axis=0, keepdims=True)


def _finalize_body(z_ref, q_ref, cnt_ref, qst_ref, loss_ref, ppl_ref, sse_ref):
    i = pl.program_id(0)

    @pl.when(i == 0)
    def _init():
        sse_ref[...] = jnp.zeros_like(sse_ref)

    z = z_ref[...]
    d = q_ref[...] - z
    qst_ref[...] = z + d
    dd = d * d
    acc = sse_ref[...]
    for s in range(FTILE // TILE):
        acc = acc + jnp.sum(dd[s * TILE:(s + 1) * TILE, :]).reshape(1, 1)
    sse_ref[...] = acc

    @pl.when(i == FGRID - 1)
    def _fin():
        loss_ref[...] = 2.0 * (sse_ref[...] / float(N * F))
        avg = cnt_ref[...] / float(N)
        ppl_ref[...] = jnp.exp(-jnp.sum(avg * jnp.log(avg + 1e-10))).reshape(1, 1)


def _make_sc_gather():
    info = plsc.get_sparse_core_info()
    nw = info.num_cores * info.num_subcores
    b_per_w = N // nw
    ch = 128
    mesh = plsc.VectorSubcoreMesh(core_axis_name="c", subcore_axis_name="s")

    @functools.partial(
        pl.kernel, mesh=mesh,
        out_type=jax.ShapeDtypeStruct((N, F), jnp.float32),
        scratch_types=[
            pltpu.VMEM((ch,), jnp.int32),
            pltpu.VMEM((ch,), jnp.int32),
            pltpu.VMEM((ch, F), jnp.float32),
            pltpu.VMEM((ch, F), jnp.float32),
            pltpu.SemaphoreType.DMA,
            pltpu.SemaphoreType.DMA,
        ],
    )
    def gather(cb_hbm, idx_hbm, q_hbm, idx_v0, idx_v1, rows_v0, rows_v1,
               sem0, sem1):
        wid = lax.axis_index("s") * info.num_cores + lax.axis_index("c")
        base = wid * b_per_w
        bufs = [(idx_v0, rows_v0, sem0), (idx_v1, rows_v1, sem1)]
        prev = None
        for c in range(b_per_w // ch):
            off = base + c * ch
            idx_v, rows_v, sem = bufs[c % 2]
            pltpu.sync_copy(idx_hbm.at[pl.ds(off, ch)], idx_v)
            fut = pltpu.async_copy(cb_hbm.at[idx_v], rows_v, sem)
            if prev is not None:
                pfut, prows, poff = prev
                pfut.wait()
                pltpu.sync_copy(prows, q_hbm.at[pl.ds(poff, ch)])
            prev = (fut, rows_v, off)
        pfut, prows, poff = prev
        pfut.wait()
        pltpu.sync_copy(prows, q_hbm.at[pl.ds(poff, ch)])

    return gather


_SC_GATHER = _make_sc_gather()

_GUMBEL = jax.random.gumbel(jax.random.key(42), (N, K), jnp.float32)


def kernel(input_data, W1, b1, W2, b2, code_book):
    g = _GUMBEL
    b1r = b1.reshape(1, HID)
    b2r = b2.reshape(1, F)
    cbt = code_book.T
    csq = jnp.sum(code_book**2, axis=1).reshape(1, K)

    row_spec = lambda w: pl.BlockSpec((TILE, w), lambda i: (i, 0))
    fin_spec = lambda w: pl.BlockSpec((FTILE, w), lambda i: (i, 0))
    full_spec = lambda a, b: pl.BlockSpec((a, b), lambda i: (0, 0))

    nd, enc, z, idx, cnt = pl.pallas_call(
        _encode_body,
        grid=(GRID,),
        in_specs=[
            row_spec(DIN),
            row_spec(K),
            full_spec(DIN, HID),
            full_spec(1, HID),
            full_spec(HID, F),
            full_spec(1, F),
            full_spec(F, K),
            full_spec(1, K),
        ],
        out_specs=(
            row_spec(K),
            row_spec(K),
            row_spec(F),
            pl.BlockSpec((TILE, 1), lambda i: (i, 0)),
            full_spec(1, K),
        ),
        out_shape=(
            jax.ShapeDtypeStruct((N, K), jnp.float32),
            jax.ShapeDtypeStruct((N, K), jnp.float32),
            jax.ShapeDtypeStruct((N, F), jnp.float32),
            jax.ShapeDtypeStruct((N, 1), jnp.int32),
            jax.ShapeDtypeStruct((1, K), jnp.float32),
        ),
    )(input_data, g, W1, b1r, W2, b2r, cbt, csq)

    q = _SC_GATHER(code_book, idx.reshape(N))

    qst, loss, ppl = pl.pallas_call(
        _finalize_body,
        grid=(FGRID,),
        in_specs=[
            fin_spec(F),
            fin_spec(F),
            full_spec(1, K),
        ],
        out_specs=(
            fin_spec(F),
            full_spec(1, 1),
            full_spec(1, 1),
        ),
        out_shape=(
            jax.ShapeDtypeStruct((N, F), jnp.float32),
            jax.ShapeDtypeStruct((1, 1), jnp.float32),
            jax.ShapeDtypeStruct((1, 1), jnp.float32),
        ),
        scratch_shapes=[
            pltpu.VMEM((1, 1), jnp.float32),
        ],
    )(z, q, cnt)

    return (loss.reshape(()), qst, ppl.reshape(()), enc, nd)


# speedup vs baseline: 1.2813x; 1.0630x over previous
import functools

import jax
import jax.numpy as jnp
from jax import lax
from jax.experimental import pallas as pl
from jax.experimental.pallas import tpu as pltpu
from jax.experimental.pallas import tpu_sc as plsc

N = 16384
DIN = 768
HID = 128
F = 256
K = 1024
TILE = 1024
GRID = N // TILE
FTILE = 2048
FGRID = N // FTILE


def _row_sq_norm(z):
    zz = z * z
    acc = zz[:, 0:8]
    for k in range(1, 32):
        acc = acc + zz[:, 8 * k:8 * k + 8]
    a = [acc[:, s:s + 1] for s in range(8)]
    return ((a[0] + a[4]) + (a[2] + a[6])) + ((a[1] + a[5]) + (a[3] + a[7]))


def _encode_body(x_ref, g_ref, w1_ref, b1_ref, w2_ref, b2_ref, cbt_ref,
                 csq_ref, nd_ref, enc_ref, z_ref, idx_ref, cnt_ref):
    i = pl.program_id(0)

    @pl.when(i == 0)
    def _init():
        cnt_ref[...] = jnp.zeros_like(cnt_ref)

    x = x_ref[...]
    h = jnp.maximum(jnp.dot(x, w1_ref[...]) + b1_ref[...], 0.0)
    z = jnp.dot(h, w2_ref[...]) + b2_ref[...]
    z_ref[...] = z

    zsq = _row_sq_norm(z)
    csq = csq_ref[...]
    zc = jnp.dot(z, cbt_ref[...])
    dist = (zsq + csq) - 2.0 * zc
    nd = -dist
    nd_ref[...] = nd
    logits = nd / 0.1
    score = g_ref[...] + logits
    idx = jnp.argmax(score, axis=1)
    idx_ref[...] = idx.reshape(TILE, 1)

    iota = jax.lax.broadcasted_iota(jnp.int32, (TILE, K), 1)
    enc = (idx[:, None] == iota).astype(jnp.float32)
    enc_ref[...] = enc
    cnt_ref[...] += jnp.sum(enc, axis=0, keepdims=True)


def _finalize_body(z_ref, q_ref, cnt_ref, qst_ref, loss_ref, ppl_ref, sse_ref):
    i = pl.program_id(0)

    @pl.when(i == 0)
    def _init():
        sse_ref[...] = jnp.zeros_like(sse_ref)

    z = z_ref[...]
    d = q_ref[...] - z
    qst_ref[...] = z + d
    dd = d * d
    acc = sse_ref[...]
    for s in range(FTILE // TILE):
        acc = acc + jnp.sum(dd[s * TILE:(s + 1) * TILE, :]).reshape(1, 1)
    sse_ref[...] = acc

    @pl.when(i == FGRID - 1)
    def _fin():
        loss_ref[...] = 2.0 * (sse_ref[...] / float(N * F))
        avg = cnt_ref[...] / float(N)
        ppl_ref[...] = jnp.exp(-jnp.sum(avg * jnp.log(avg + 1e-10))).reshape(1, 1)


def _make_sc_gather():
    info = plsc.get_sparse_core_info()
    nw = info.num_cores * info.num_subcores
    b_per_w = N // nw
    ch = 128
    mesh = plsc.VectorSubcoreMesh(core_axis_name="c", subcore_axis_name="s")

    @functools.partial(
        pl.kernel, mesh=mesh,
        out_type=jax.ShapeDtypeStruct((N, F), jnp.float32),
        scratch_types=[
            pltpu.VMEM((ch,), jnp.int32),
            pltpu.VMEM((ch,), jnp.int32),
            pltpu.VMEM((ch, F), jnp.float32),
            pltpu.VMEM((ch, F), jnp.float32),
            pltpu.SemaphoreType.DMA,
            pltpu.SemaphoreType.DMA,
        ],
    )
    def gather(cb_hbm, idx_hbm, q_hbm, idx_v0, idx_v1, rows_v0, rows_v1,
               sem0, sem1):
        wid = lax.axis_index("s") * info.num_cores + lax.axis_index("c")
        base = wid * b_per_w
        bufs = [(idx_v0, rows_v0, sem0), (idx_v1, rows_v1, sem1)]
        prev = None
        for c in range(b_per_w // ch):
            off = base + c * ch
            idx_v, rows_v, sem = bufs[c % 2]
            pltpu.sync_copy(idx_hbm.at[pl.ds(off, ch)], idx_v)
            fut = pltpu.async_copy(cb_hbm.at[idx_v], rows_v, sem)
            if prev is not None:
                pfut, prows, poff = prev
                pfut.wait()
                pltpu.sync_copy(prows, q_hbm.at[pl.ds(poff, ch)])
            prev = (fut, rows_v, off)
        pfut, prows, poff = prev
        pfut.wait()
        pltpu.sync_copy(prows, q_hbm.at[pl.ds(poff, ch)])

    return gather


_SC_GATHER = _make_sc_gather()

_GUMBEL = jax.random.gumbel(jax.random.key(42), (N, K), jnp.float32)


def kernel(input_data, W1, b1, W2, b2, code_book):
    g = _GUMBEL
    b1r = b1.reshape(1, HID)
    b2r = b2.reshape(1, F)
    cbt = code_book.T
    csq = jnp.sum(code_book**2, axis=1).reshape(1, K)

    row_spec = lambda w: pl.BlockSpec((TILE, w), lambda i: (i, 0))
    fin_spec = lambda w: pl.BlockSpec((FTILE, w), lambda i: (i, 0))
    full_spec = lambda a, b: pl.BlockSpec((a, b), lambda i: (0, 0))

    nd, enc, z, idx, cnt = pl.pallas_call(
        _encode_body,
        grid=(GRID,),
        in_specs=[
            row_spec(DIN),
            row_spec(K),
            full_spec(DIN, HID),
            full_spec(1, HID),
            full_spec(HID, F),
            full_spec(1, F),
            full_spec(F, K),
            full_spec(1, K),
        ],
        out_specs=(
            row_spec(K),
            row_spec(K),
            row_spec(F),
            pl.BlockSpec((TILE, 1), lambda i: (i, 0)),
            full_spec(1, K),
        ),
        out_shape=(
            jax.ShapeDtypeStruct((N, K), jnp.float32),
            jax.ShapeDtypeStruct((N, K), jnp.float32),
            jax.ShapeDtypeStruct((N, F), jnp.float32),
            jax.ShapeDtypeStruct((N, 1), jnp.int32),
            jax.ShapeDtypeStruct((1, K), jnp.float32),
        ),
    )(input_data, g, W1, b1r, W2, b2r, cbt, csq)

    q = _SC_GATHER(code_book, idx.reshape(N))

    qst, loss, ppl = pl.pallas_call(
        _finalize_body,
        grid=(FGRID,),
        in_specs=[
            fin_spec(F),
            fin_spec(F),
            full_spec(1, K),
        ],
        out_specs=(
            fin_spec(F),
            full_spec(1, 1),
            full_spec(1, 1),
        ),
        out_shape=(
            jax.ShapeDtypeStruct((N, F), jnp.float32),
            jax.ShapeDtypeStruct((1, 1), jnp.float32),
            jax.ShapeDtypeStruct((1, 1), jnp.float32),
        ),
        scratch_shapes=[
            pltpu.VMEM((1, 1), jnp.float32),
        ],
    )(z, q, cnt)

    return (loss.reshape(()), qst, ppl.reshape(()), enc, nd)


# speedup vs baseline: 1.2856x; 1.0034x over previous
import functools

import jax
import jax.numpy as jnp
from jax import lax
from jax.experimental import pallas as pl
from jax.experimental.pallas import tpu as pltpu
from jax.experimental.pallas import tpu_sc as plsc

N = 16384
DIN = 768
HID = 128
F = 256
K = 1024
TILE = 1024
GRID = N // TILE
FTILE = 4096
FGRID = N // FTILE


def _row_sq_norm(z):
    zz = z * z
    acc = zz[:, 0:8]
    for k in range(1, 32):
        acc = acc + zz[:, 8 * k:8 * k + 8]
    a = [acc[:, s:s + 1] for s in range(8)]
    return ((a[0] + a[4]) + (a[2] + a[6])) + ((a[1] + a[5]) + (a[3] + a[7]))


def _encode_body(x_ref, g_ref, w1_ref, b1_ref, w2_ref, b2_ref, cbt_ref,
                 csq_ref, nd_ref, enc_ref, z_ref, idx_ref, cnt_ref):
    i = pl.program_id(0)

    @pl.when(i == 0)
    def _init():
        cnt_ref[...] = jnp.zeros_like(cnt_ref)

    x = x_ref[...]
    h = jnp.maximum(jnp.dot(x, w1_ref[...]) + b1_ref[...], 0.0)
    z = jnp.dot(h, w2_ref[...]) + b2_ref[...]
    z_ref[...] = z

    zsq = _row_sq_norm(z)
    csq = csq_ref[...]
    zc = jnp.dot(z, cbt_ref[...])
    dist = (zsq + csq) - 2.0 * zc
    nd = -dist
    nd_ref[...] = nd
    logits = nd / 0.1
    score = g_ref[...] + logits
    idx = jnp.argmax(score, axis=1)
    idx_ref[...] = idx.reshape(TILE, 1)

    iota = jax.lax.broadcasted_iota(jnp.int32, (TILE, K), 1)
    enc = (idx[:, None] == iota).astype(jnp.float32)
    enc_ref[...] = enc
    cnt_ref[...] += jnp.sum(enc, axis=0, keepdims=True)


def _finalize_body(z_ref, q_ref, cnt_ref, qst_ref, loss_ref, ppl_ref, sse_ref):
    i = pl.program_id(0)

    @pl.when(i == 0)
    def _init():
        sse_ref[...] = jnp.zeros_like(sse_ref)

    z = z_ref[...]
    d = q_ref[...] - z
    qst_ref[...] = z + d
    dd = d * d
    acc = sse_ref[...]
    for s in range(FTILE // TILE):
        acc = acc + jnp.sum(dd[s * TILE:(s + 1) * TILE, :]).reshape(1, 1)
    sse_ref[...] = acc

    @pl.when(i == FGRID - 1)
    def _fin():
        loss_ref[...] = 2.0 * (sse_ref[...] / float(N * F))
        avg = cnt_ref[...] / float(N)
        ppl_ref[...] = jnp.exp(-jnp.sum(avg * jnp.log(avg + 1e-10))).reshape(1, 1)


def _make_sc_gather():
    info = plsc.get_sparse_core_info()
    nw = info.num_cores * info.num_subcores
    b_per_w = N // nw
    ch = 128
    mesh = plsc.VectorSubcoreMesh(core_axis_name="c", subcore_axis_name="s")

    @functools.partial(
        pl.kernel, mesh=mesh,
        out_type=jax.ShapeDtypeStruct((N, F), jnp.float32),
        scratch_types=[
            pltpu.VMEM((ch,), jnp.int32),
            pltpu.VMEM((ch,), jnp.int32),
            pltpu.VMEM((ch, F), jnp.float32),
            pltpu.VMEM((ch, F), jnp.float32),
            pltpu.SemaphoreType.DMA,
            pltpu.SemaphoreType.DMA,
        ],
    )
    def gather(cb_hbm, idx_hbm, q_hbm, idx_v0, idx_v1, rows_v0, rows_v1,
               sem0, sem1):
        wid = lax.axis_index("s") * info.num_cores + lax.axis_index("c")
        base = wid * b_per_w
        bufs = [(idx_v0, rows_v0, sem0), (idx_v1, rows_v1, sem1)]
        prev = None
        for c in range(b_per_w // ch):
            off = base + c * ch
            idx_v, rows_v, sem = bufs[c % 2]
            pltpu.sync_copy(idx_hbm.at[pl.ds(off, ch)], idx_v)
            fut = pltpu.async_copy(cb_hbm.at[idx_v], rows_v, sem)
            if prev is not None:
                pfut, prows, poff = prev
                pfut.wait()
                pltpu.sync_copy(prows, q_hbm.at[pl.ds(poff, ch)])
            prev = (fut, rows_v, off)
        pfut, prows, poff = prev
        pfut.wait()
        pltpu.sync_copy(prows, q_hbm.at[pl.ds(poff, ch)])

    return gather


_SC_GATHER = _make_sc_gather()

_GUMBEL = jax.random.gumbel(jax.random.key(42), (N, K), jnp.float32)


def kernel(input_data, W1, b1, W2, b2, code_book):
    g = _GUMBEL
    b1r = b1.reshape(1, HID)
    b2r = b2.reshape(1, F)
    cbt = code_book.T
    csq = jnp.sum(code_book**2, axis=1).reshape(1, K)

    row_spec = lambda w: pl.BlockSpec((TILE, w), lambda i: (i, 0))
    fin_spec = lambda w: pl.BlockSpec((FTILE, w), lambda i: (i, 0))
    full_spec = lambda a, b: pl.BlockSpec((a, b), lambda i: (0, 0))

    nd, enc, z, idx, cnt = pl.pallas_call(
        _encode_body,
        grid=(GRID,),
        in_specs=[
            row_spec(DIN),
            row_spec(K),
            full_spec(DIN, HID),
            full_spec(1, HID),
            full_spec(HID, F),
            full_spec(1, F),
            full_spec(F, K),
            full_spec(1, K),
        ],
        out_specs=(
            row_spec(K),
            row_spec(K),
            row_spec(F),
            pl.BlockSpec((TILE, 1), lambda i: (i, 0)),
            full_spec(1, K),
        ),
        out_shape=(
            jax.ShapeDtypeStruct((N, K), jnp.float32),
            jax.ShapeDtypeStruct((N, K), jnp.float32),
            jax.ShapeDtypeStruct((N, F), jnp.float32),
            jax.ShapeDtypeStruct((N, 1), jnp.int32),
            jax.ShapeDtypeStruct((1, K), jnp.float32),
        ),
    )(input_data, g, W1, b1r, W2, b2r, cbt, csq)

    q = _SC_GATHER(code_book, idx.reshape(N))

    qst, loss, ppl = pl.pallas_call(
        _finalize_body,
        grid=(FGRID,),
        in_specs=[
            fin_spec(F),
            fin_spec(F),
            full_spec(1, K),
        ],
        out_specs=(
            fin_spec(F),
            full_spec(1, 1),
            full_spec(1, 1),
        ),
        out_shape=(
            jax.ShapeDtypeStruct((N, F), jnp.float32),
            jax.ShapeDtypeStruct((1, 1), jnp.float32),
            jax.ShapeDtypeStruct((1, 1), jnp.float32),
        ),
        scratch_shapes=[
            pltpu.VMEM((1, 1), jnp.float32),
        ],
    )(z, q, cnt)

    return (loss.reshape(()), qst, ppl.reshape(()), enc, nd)
